# baseline (speedup 1.0000x reference)
import jax
import jax.numpy as jnp
from jax import lax
from jax.experimental import pallas as pl

N = 16384
F = 26
V = 100000
D = 64
S = 768
K = F * D

_BN = 512


def _mm_body(e_ref, s_ref, w_ref, b_ref, o_ref):
    g = jnp.concatenate([e_ref[f] for f in range(F)], axis=1)
    acc = lax.dot_general(
        g, w_ref[...],
        (((1,), (1,)), ((), ())),
        preferred_element_type=jnp.float32,
    )
    o_ref[:, :S] = s_ref[...]
    o_ref[:, S:] = acc + b_ref[...]


_CH = 4
_NCH = N // _CH


def kernel(sentence_embeddings, categorical_data, tables, W, b):
    b2 = b.reshape(1, S)
    outs = []
    for c in range(_CH):
        sl = slice(c * _NCH, (c + 1) * _NCH)
        emb_c = jax.vmap(lambda t, i: jnp.take(t, i, axis=0))(
            tables, categorical_data[:, sl])
        out_c = pl.pallas_call(
            _mm_body,
            grid=(_NCH // _BN,),
            in_specs=[
                pl.BlockSpec((F, _BN, D), lambda i: (0, i, 0)),
                pl.BlockSpec((_BN, S), lambda i: (i, 0)),
                pl.BlockSpec((S, K), lambda i: (0, 0)),
                pl.BlockSpec((1, S), lambda i: (0, 0)),
            ],
            out_specs=pl.BlockSpec((_BN, 2 * S), lambda i: (i, 0)),
            out_shape=jax.ShapeDtypeStruct((_NCH, 2 * S), jnp.float32),
        )(emb_c, sentence_embeddings[sl], W, b2)
        outs.append(out_c)
    return jnp.concatenate(outs, axis=0)


# speedup vs baseline: 1.1383x; 1.1383x over previous
import jax
import jax.numpy as jnp
from jax import lax
from jax.experimental import pallas as pl

N = 16384
F = 26
V = 100000
D = 64
S = 768
K = F * D

_BN = 512


def _mm_body(e_ref, s_ref, w_ref, b_ref, o_ref):
    g = jnp.concatenate([e_ref[f] for f in range(F)], axis=1)
    acc = lax.dot_general(
        g, w_ref[...],
        (((1,), (1,)), ((), ())),
        preferred_element_type=jnp.float32,
    )
    o_ref[:, :S] = s_ref[...]
    o_ref[:, S:] = acc + b_ref[...]


def kernel(sentence_embeddings, categorical_data, tables, W, b):
    emb = jax.vmap(lambda t, i: jnp.take(t, i, axis=0))(
        tables, categorical_data)
    out = pl.pallas_call(
        _mm_body,
        grid=(N // _BN,),
        in_specs=[
            pl.BlockSpec((F, _BN, D), lambda i: (0, i, 0)),
            pl.BlockSpec((_BN, S), lambda i: (i, 0)),
            pl.BlockSpec((S, K), lambda i: (0, 0)),
            pl.BlockSpec((1, S), lambda i: (0, 0)),
        ],
        out_specs=pl.BlockSpec((_BN, 2 * S), lambda i: (i, 0)),
        out_shape=jax.ShapeDtypeStruct((N, 2 * S), jnp.float32),
    )(emb, sentence_embeddings, W, b.reshape(1, S))
    return out
